# baseline (speedup 1.0000x reference)
import jax
import jax.numpy as jnp
from jax import lax
from jax.experimental import pallas as pl
from jax.experimental.pallas import tpu as pltpu
from jax.experimental.pallas import tpu_sc as plsc

N = 100000
E2 = 1700000
BINW = 256
NBIN = 416
NPB = 13
NPAD = NBIN * BINW
CAPB = 4864
NBLKB = CAPB // 128
ROWW = 256
VROWS = BINW + 8


def _sc_body(p_hbm, ad_hbm, psrc_hbm, pdstl_hbm, out_hbm,
             ad_v, src_v, dstl_v, rows_v, vbuf, sem):
    core = lax.axis_index("c")
    sid = lax.axis_index("s")
    wid = core * 16 + sid
    lane = lax.iota(jnp.int32, 16)
    ad_v[pl.ds(BINW * 8, 16)] = jnp.zeros((16,), jnp.float32)

    zero16 = jnp.zeros((16,), jnp.float32)

    def bin_body(p, _):
        bin_ = p * 32 + wid

        pltpu.sync_copy(ad_hbm.at[pl.ds(bin_ * (BINW * 8), BINW * 8)],
                        ad_v.at[pl.ds(0, BINW * 8)])

        ebase = bin_ * CAPB

        def blk(b, car):
            eoff = ebase + b * 128
            pltpu.sync_copy(psrc_hbm.at[pl.ds(eoff, 128)], src_v)
            pltpu.sync_copy(pdstl_hbm.at[pl.ds(eoff, 128)], dstl_v)
            pltpu.async_copy(p_hbm.at[src_v], rows_v, sem).wait()

            def grp(g, car):
                dv = dstl_v[pl.ds(pl.multiple_of(g * 16, 16), 16)]
                for i in range(16):
                    er = g * 16 + i
                    dl = dv[i]
                    prev = car[0]
                    changed = dl != prev
                    accs = [jnp.where(changed, zero16, car[1 + k])
                            for k in range(13)]

                    av = rows_v[er, pl.ds(192, 16)]
                    dmin = jnp.minimum(dl, BINW - 1)
                    adv = ad_v[pl.ds(pl.multiple_of(dmin * 8, 8), 16)]
                    e = av + adv
                    e = jnp.maximum(e, 0.2 * e)
                    wv = jnp.exp(e)
                    new = []
                    for h in range(3):
                        wh = jnp.full((16,), wv[h])
                        for cb in range(4):
                            k = h * 4 + cb
                            off = h * 64 + cb * 16
                            a = accs[k] + rows_v[er, pl.ds(off, 16)] * wh
                            vbuf[dl, pl.ds(off, 16)] = a
                            new.append(a)
                    den = accs[12] + jnp.where(lane < 3, wv, zero16)
                    vbuf[dl, pl.ds(192, 16)] = den
                    car = (dl,) + tuple(new) + (den,)
                return car
            return lax.fori_loop(0, 8, grp, car)
        init = (jnp.int32(BINW),) + (zero16,) * 13
        lax.fori_loop(0, NBLKB, blk, init)

        def frow(r, _):
            dvv = vbuf[r, pl.ds(192, 16)]
            recv = 1.0 / (dvv + 1e-16)
            for h in range(3):
                rec = jnp.full((16,), recv[h])
                for cb in range(4):
                    off = h * 64 + cb * 16
                    vbuf[r, pl.ds(off, 16)] = vbuf[r, pl.ds(off, 16)] * rec
            return ()
        lax.fori_loop(0, BINW, frow, ())
        pltpu.sync_copy(vbuf.at[pl.ds(0, BINW)],
                        out_hbm.at[pl.ds(bin_ * BINW, BINW)])
        return ()
    lax.fori_loop(0, NPB, bin_body, ())


_sc_layer = pl.kernel(
    _sc_body,
    out_type=jax.ShapeDtypeStruct((NPAD, ROWW), jnp.float32),
    mesh=plsc.VectorSubcoreMesh(
        core_axis_name="c", subcore_axis_name="s",
        num_cores=2, num_subcores=16),
    scratch_types=[
        pltpu.VMEM((BINW * 8 + 16,), jnp.float32),
        pltpu.VMEM((128,), jnp.int32),
        pltpu.VMEM((128,), jnp.int32),
        pltpu.VMEM((128, ROWW), jnp.float32),
        pltpu.VMEM((VROWS, ROWW), jnp.float32),
        pltpu.SemaphoreType.DMA,
    ],
)


def _tc_first(xp, W1, Asd, Ad8):
    def body(x_ref, w_ref, asd_ref, ad_ref, p_ref, adout_ref):
        h = x_ref[...] @ w_ref[...]
        p_ref[:, 0:192] = h
        p_ref[:, 192:256] = h @ asd_ref[...]
        adout_ref[...] = h @ ad_ref[...]

    return pl.pallas_call(
        body,
        grid=(NPAD // 512,),
        in_specs=[
            pl.BlockSpec((512, 12), lambda i: (i, 0)),
            pl.BlockSpec((12, 192), lambda i: (0, 0)),
            pl.BlockSpec((192, 64), lambda i: (0, 0)),
            pl.BlockSpec((192, 8), lambda i: (0, 0)),
        ],
        out_specs=[
            pl.BlockSpec((512, ROWW), lambda i: (i, 0)),
            pl.BlockSpec((512, 8), lambda i: (i, 0)),
        ],
        out_shape=[
            jax.ShapeDtypeStruct((NPAD, ROWW), jnp.float32),
            jax.ShapeDtypeStruct((NPAD, 8), jnp.float32),
        ],
    )(xp, W1, Asd, Ad8)


def _tc_mid(y, b, W2, Asd, Ad8):
    def body(y_ref, b_ref, w_ref, asd_ref, ad_ref, p_ref, adout_ref):
        h = (y_ref[:, 0:192] + b_ref[...]) @ w_ref[...]
        p_ref[:, 0:192] = h
        p_ref[:, 192:256] = h @ asd_ref[...]
        adout_ref[...] = h @ ad_ref[...]

    return pl.pallas_call(
        body,
        grid=(NPAD // 512,),
        in_specs=[
            pl.BlockSpec((512, ROWW), lambda i: (i, 0)),
            pl.BlockSpec((1, 192), lambda i: (0, 0)),
            pl.BlockSpec((192, 192), lambda i: (0, 0)),
            pl.BlockSpec((192, 64), lambda i: (0, 0)),
            pl.BlockSpec((192, 8), lambda i: (0, 0)),
        ],
        out_specs=[
            pl.BlockSpec((512, ROWW), lambda i: (i, 0)),
            pl.BlockSpec((512, 8), lambda i: (i, 0)),
        ],
        out_shape=[
            jax.ShapeDtypeStruct((NPAD, ROWW), jnp.float32),
            jax.ShapeDtypeStruct((NPAD, 8), jnp.float32),
        ],
    )(y, b, W2, Asd, Ad8)


def _tc_last(y, b, Wl, bl):
    def body(y_ref, b_ref, wl_ref, bl_ref, o_ref):
        o_ref[...] = (y_ref[:, 0:192] + b_ref[...]) @ wl_ref[...] + bl_ref[...]

    return pl.pallas_call(
        body,
        grid=(100,),
        in_specs=[
            pl.BlockSpec((1000, ROWW), lambda i: (i, 0)),
            pl.BlockSpec((1, 192), lambda i: (0, 0)),
            pl.BlockSpec((192, 1), lambda i: (0, 0)),
            pl.BlockSpec((1, 1), lambda i: (0, 0)),
        ],
        out_specs=pl.BlockSpec((1000, 1), lambda i: (i, 0)),
        out_shape=jax.ShapeDtypeStruct((N, 1), jnp.float32),
    )(y, b, Wl, bl)


def kernel(x, edge_index, W1, a1s, a1d, b1, W2, a2s, a2d, b2, Wl, bl):
    i32 = jnp.int32
    loops = jnp.arange(N, dtype=edge_index.dtype)
    src_all = jnp.concatenate([edge_index[0], loops]).astype(i32)
    dst_all = jnp.concatenate([edge_index[1], loops]).astype(i32)

    order = jnp.argsort(dst_all)
    sd = dst_all[order]
    ss = src_all[order]
    starts = jnp.searchsorted(
        sd, jnp.arange(NBIN + 1, dtype=i32) * BINW).astype(i32)
    slot = jnp.arange(NBIN * CAPB, dtype=i32)
    c = slot // CAPB
    j = slot % CAPB
    take = starts[c] + j
    valid = take < starts[c + 1]
    takec = jnp.minimum(take, E2 - 1)
    psrc = jnp.where(valid, ss[takec], 0)
    pdstl = jnp.where(valid, sd[takec] - c * BINW, BINW + (slot & 7))

    heads = jnp.repeat(jnp.arange(3, dtype=i32), 64)
    r192 = jnp.arange(192, dtype=i32)

    def aug(a_s, a_d):
        Asd = jnp.zeros((192, 64), jnp.float32).at[r192, heads].set(
            a_s.reshape(192))
        Ad8 = jnp.zeros((192, 8), jnp.float32).at[r192, heads].set(
            a_d.reshape(192))
        return Asd, Ad8

    Asd1, Ad81 = aug(a1s, a1d)
    Asd2, Ad82 = aug(a2s, a2d)

    xp = jnp.zeros((NPAD, 12), jnp.float32).at[:N].set(x)
    P1, AD1 = _tc_first(xp, W1, Asd1, Ad81)
    Y1 = _sc_layer(P1, AD1.reshape(NPAD * 8), psrc, pdstl)
    P2, AD2 = _tc_mid(Y1, b1.reshape(1, 192), W2, Asd2, Ad82)
    Y2 = _sc_layer(P2, AD2.reshape(NPAD * 8), psrc, pdstl)
    return _tc_last(Y2, b2.reshape(1, 192), Wl, bl.reshape(1, 1))


# speedup vs baseline: 1.0413x; 1.0413x over previous
import jax
import jax.numpy as jnp
from jax import lax
from jax.experimental import pallas as pl
from jax.experimental.pallas import tpu as pltpu
from jax.experimental.pallas import tpu_sc as plsc

N = 100000
E2 = 1700000
BINW = 192
NBIN = 544
NPB = 17
NPAD = NBIN * BINW
CAPB = 3840
NBLKB = CAPB // 128
NB2 = NBLKB // 2
ROWW = 256
OUTW = 208
VROWS = BINW + 8


def _sc_body(p_hbm, ad_hbm, psrc_hbm, pdstl_hbm, out_hbm,
             ad_v, srcs_v, dstls_v, rows0_v, rows1_v, vbuf, sem0, sem1):
    core = lax.axis_index("c")
    sid = lax.axis_index("s")
    wid = core * 16 + sid
    lane = lax.iota(jnp.int32, 16)
    ad_v[pl.ds(BINW * 8, 16)] = jnp.zeros((16,), jnp.float32)

    zero16 = jnp.zeros((16,), jnp.float32)

    def gather(b, rows_v, sem):
        return pltpu.async_copy(
            p_hbm.at[srcs_v.at[pl.ds(b * 128, 128)]], rows_v, sem)

    def bin_body(p, _):
        bin_ = p * 32 + wid

        pltpu.sync_copy(ad_hbm.at[pl.ds(bin_ * (BINW * 8), BINW * 8)],
                        ad_v.at[pl.ds(0, BINW * 8)])
        ebase = bin_ * CAPB
        pltpu.sync_copy(psrc_hbm.at[pl.ds(ebase, CAPB)], srcs_v)
        pltpu.sync_copy(pdstl_hbm.at[pl.ds(ebase, CAPB)], dstls_v)

        def compute(b, rows_v, car):
            def grp(g, car):
                dv = dstls_v[
                    pl.ds(pl.multiple_of(b * 128 + g * 16, 16), 16)]
                for i in range(16):
                    er = g * 16 + i
                    dl = dv[i]
                    prev = car[0]
                    changed = dl != prev
                    accs = [jnp.where(changed, zero16, car[1 + k])
                            for k in range(13)]

                    av = rows_v[er, pl.ds(192, 16)]
                    dmin = jnp.minimum(dl, BINW - 1)
                    adv = ad_v[pl.ds(pl.multiple_of(dmin * 8, 8), 16)]
                    e = av + adv
                    e = jnp.maximum(e, 0.2 * e)
                    wv = jnp.exp(e)
                    new = []
                    for h in range(3):
                        wh = jnp.full((16,), wv[h])
                        for cb in range(4):
                            k = h * 4 + cb
                            off = h * 64 + cb * 16
                            a = accs[k] + rows_v[er, pl.ds(off, 16)] * wh
                            vbuf[dl, pl.ds(off, 16)] = a
                            new.append(a)
                    den = accs[12] + jnp.where(lane < 3, wv, zero16)
                    vbuf[dl, pl.ds(192, 16)] = den
                    car = (dl,) + tuple(new) + (den,)
                return car
            return lax.fori_loop(0, 8, grp, car)

        gather(0, rows0_v, sem0)
        gather(1, rows1_v, sem1)

        def blk2(t, car):
            b0 = t * 2
            pltpu.make_async_copy(
                p_hbm.at[srcs_v.at[pl.ds(0, 128)]], rows0_v, sem0).wait()
            car = compute(b0, rows0_v, car)

            @pl.when(t + 1 < NB2)
            def _():
                gather(b0 + 2, rows0_v, sem0)

            pltpu.make_async_copy(
                p_hbm.at[srcs_v.at[pl.ds(0, 128)]], rows1_v, sem1).wait()
            car = compute(b0 + 1, rows1_v, car)

            @pl.when(t + 1 < NB2)
            def _():
                gather(b0 + 3, rows1_v, sem1)
            return car
        init = (jnp.int32(BINW),) + (zero16,) * 13
        lax.fori_loop(0, NB2, blk2, init)

        def frow(r, _):
            dvv = vbuf[r, pl.ds(192, 16)]
            recv = 1.0 / (dvv + 1e-16)
            for h in range(3):
                rec = jnp.full((16,), recv[h])
                for cb in range(4):
                    off = h * 64 + cb * 16
                    vbuf[r, pl.ds(off, 16)] = vbuf[r, pl.ds(off, 16)] * rec
            return ()
        lax.fori_loop(0, BINW, frow, ())
        pltpu.sync_copy(vbuf.at[pl.ds(0, BINW)],
                        out_hbm.at[pl.ds(bin_ * BINW, BINW)])
        return ()
    lax.fori_loop(0, NPB, bin_body, ())


_sc_layer = pl.kernel(
    _sc_body,
    out_type=jax.ShapeDtypeStruct((NPAD, OUTW), jnp.float32),
    mesh=plsc.VectorSubcoreMesh(
        core_axis_name="c", subcore_axis_name="s",
        num_cores=2, num_subcores=16),
    scratch_types=[
        pltpu.VMEM((BINW * 8 + 16,), jnp.float32),
        pltpu.VMEM((CAPB,), jnp.int32),
        pltpu.VMEM((CAPB,), jnp.int32),
        pltpu.VMEM((128, ROWW), jnp.float32),
        pltpu.VMEM((128, ROWW), jnp.float32),
        pltpu.VMEM((VROWS, OUTW), jnp.float32),
        pltpu.SemaphoreType.DMA,
        pltpu.SemaphoreType.DMA,
    ],
)


def _tc_first(xp, W1, Asd, Ad8):
    def body(x_ref, w_ref, asd_ref, ad_ref, p_ref, adout_ref):
        h = x_ref[...] @ w_ref[...]
        p_ref[:, 0:192] = h
        p_ref[:, 192:256] = h @ asd_ref[...]
        adout_ref[...] = h @ ad_ref[...]

    return pl.pallas_call(
        body,
        grid=(NPAD // 512,),
        in_specs=[
            pl.BlockSpec((512, 12), lambda i: (i, 0)),
            pl.BlockSpec((12, 192), lambda i: (0, 0)),
            pl.BlockSpec((192, 64), lambda i: (0, 0)),
            pl.BlockSpec((192, 8), lambda i: (0, 0)),
        ],
        out_specs=[
            pl.BlockSpec((512, ROWW), lambda i: (i, 0)),
            pl.BlockSpec((512, 8), lambda i: (i, 0)),
        ],
        out_shape=[
            jax.ShapeDtypeStruct((NPAD, ROWW), jnp.float32),
            jax.ShapeDtypeStruct((NPAD, 8), jnp.float32),
        ],
    )(xp, W1, Asd, Ad8)


def _tc_mid(y, b, W2, Asd, Ad8):
    def body(y_ref, b_ref, w_ref, asd_ref, ad_ref, p_ref, adout_ref):
        h = (y_ref[:, 0:192] + b_ref[...]) @ w_ref[...]
        p_ref[:, 0:192] = h
        p_ref[:, 192:256] = h @ asd_ref[...]
        adout_ref[...] = h @ ad_ref[...]

    return pl.pallas_call(
        body,
        grid=(NPAD // 512,),
        in_specs=[
            pl.BlockSpec((512, OUTW), lambda i: (i, 0)),
            pl.BlockSpec((1, 192), lambda i: (0, 0)),
            pl.BlockSpec((192, 192), lambda i: (0, 0)),
            pl.BlockSpec((192, 64), lambda i: (0, 0)),
            pl.BlockSpec((192, 8), lambda i: (0, 0)),
        ],
        out_specs=[
            pl.BlockSpec((512, ROWW), lambda i: (i, 0)),
            pl.BlockSpec((512, 8), lambda i: (i, 0)),
        ],
        out_shape=[
            jax.ShapeDtypeStruct((NPAD, ROWW), jnp.float32),
            jax.ShapeDtypeStruct((NPAD, 8), jnp.float32),
        ],
    )(y, b, W2, Asd, Ad8)


def _tc_last(y, b, Wl, bl):
    def body(y_ref, b_ref, wl_ref, bl_ref, o_ref):
        o_ref[...] = (y_ref[:, 0:192] + b_ref[...]) @ wl_ref[...] + bl_ref[...]

    return pl.pallas_call(
        body,
        grid=(100,),
        in_specs=[
            pl.BlockSpec((1000, OUTW), lambda i: (i, 0)),
            pl.BlockSpec((1, 192), lambda i: (0, 0)),
            pl.BlockSpec((192, 1), lambda i: (0, 0)),
            pl.BlockSpec((1, 1), lambda i: (0, 0)),
        ],
        out_specs=pl.BlockSpec((1000, 1), lambda i: (i, 0)),
        out_shape=jax.ShapeDtypeStruct((N, 1), jnp.float32),
    )(y, b, Wl, bl)


def kernel(x, edge_index, W1, a1s, a1d, b1, W2, a2s, a2d, b2, Wl, bl):
    i32 = jnp.int32
    loops = jnp.arange(N, dtype=edge_index.dtype)
    src_all = jnp.concatenate([edge_index[0], loops]).astype(i32)
    dst_all = jnp.concatenate([edge_index[1], loops]).astype(i32)

    order = jnp.argsort(dst_all)
    sd = dst_all[order]
    ss = src_all[order]
    starts = jnp.searchsorted(
        sd, jnp.arange(NBIN + 1, dtype=i32) * BINW).astype(i32)
    slot = jnp.arange(NBIN * CAPB, dtype=i32)
    c = slot // CAPB
    j = slot % CAPB
    take = starts[c] + j
    valid = take < starts[c + 1]
    takec = jnp.minimum(take, E2 - 1)
    psrc = jnp.where(valid, ss[takec], 0)
    pdstl = jnp.where(valid, sd[takec] - c * BINW, BINW + (slot & 7))

    heads = jnp.repeat(jnp.arange(3, dtype=i32), 64)
    r192 = jnp.arange(192, dtype=i32)

    def aug(a_s, a_d):
        Asd = jnp.zeros((192, 64), jnp.float32).at[r192, heads].set(
            a_s.reshape(192))
        Ad8 = jnp.zeros((192, 8), jnp.float32).at[r192, heads].set(
            a_d.reshape(192))
        return Asd, Ad8

    Asd1, Ad81 = aug(a1s, a1d)
    Asd2, Ad82 = aug(a2s, a2d)

    xp = jnp.zeros((NPAD, 12), jnp.float32).at[:N].set(x)
    P1, AD1 = _tc_first(xp, W1, Asd1, Ad81)
    Y1 = _sc_layer(P1, AD1.reshape(NPAD * 8), psrc, pdstl)
    P2, AD2 = _tc_mid(Y1, b1.reshape(1, 192), W2, Asd2, Ad82)
    Y2 = _sc_layer(P2, AD2.reshape(NPAD * 8), psrc, pdstl)
    return _tc_last(Y2, b2.reshape(1, 192), Wl, bl.reshape(1, 1))
